# Initial kernel scaffold; baseline (speedup 1.0000x reference)
import jax
import jax.numpy as jnp
from jax.experimental import pallas as pl


def kernel(sequence, lookup):
    raise NotImplementedError("write your pallas kernel here")


# speedup vs baseline: 3.1728x; 3.1728x over previous
import functools

import jax
import jax.numpy as jnp
from jax import lax
from jax.experimental import pallas as pl
from jax.experimental.pallas import tpu as pltpu
from jax.experimental.pallas import tpu_sc as plsc

VOCAB = 100000
D_MODEL = 64

_NC = 2
_NS = 16
_NW = _NC * _NS

_B = 4096 * 200
_B_PER_W = _B // _NW
_CHUNK = 128
_N_CHUNK = _B_PER_W // _CHUNK


def _emb_body(idx_hbm, table_hbm, out_hbm, idx_v, rows_v, sem):
    wid = lax.axis_index("s") * _NC + lax.axis_index("c")
    base = wid * _B_PER_W

    def body(c, carry):
        off = base + c * _CHUNK
        pltpu.sync_copy(idx_hbm.at[pl.ds(off, _CHUNK)], idx_v)
        pltpu.async_copy(table_hbm.at[idx_v], rows_v, sem).wait()
        pltpu.sync_copy(rows_v, out_hbm.at[pl.ds(off, _CHUNK)])
        return carry

    lax.fori_loop(0, _N_CHUNK, body, 0)


_emb = functools.partial(
    pl.kernel,
    out_type=jax.ShapeDtypeStruct((_B, D_MODEL), jnp.float32),
    mesh=plsc.VectorSubcoreMesh(core_axis_name="c", subcore_axis_name="s"),
    scratch_types=[
        pltpu.VMEM((_CHUNK,), jnp.int32),
        pltpu.VMEM((_CHUNK, D_MODEL), jnp.float32),
        pltpu.SemaphoreType.DMA,
    ],
    compiler_params=pltpu.CompilerParams(use_tc_tiling_on_sc=False),
)(_emb_body)


def kernel(sequence, lookup):
    idx = sequence.reshape(-1).astype(jnp.int32)
    out = _emb(idx, lookup)
    return out.reshape(sequence.shape + (D_MODEL,))


# speedup vs baseline: 4.2538x; 1.3407x over previous
import functools

import jax
import jax.numpy as jnp
from jax import lax
from jax.experimental import pallas as pl
from jax.experimental.pallas import tpu as pltpu
from jax.experimental.pallas import tpu_sc as plsc

VOCAB = 100000
D_MODEL = 64

_NC = 2
_NS = 16
_NW = _NC * _NS

_B = 4096 * 200
_B_PER_W = _B // _NW
_CHUNK = 256
_N_CHUNK = _B_PER_W // _CHUNK
_NBUF = 4
_N_OUTER = _N_CHUNK // _NBUF


def _emb_body(idx_hbm, table_hbm, out_hbm, idx_v, rows_v, gsem, osem):
    wid = lax.axis_index("s") * _NC + lax.axis_index("c")
    base = wid * _B_PER_W

    def start_gather(c, b):
        off = base + c * _CHUNK
        pltpu.sync_copy(idx_hbm.at[pl.ds(off, _CHUNK)], idx_v.at[b])
        pltpu.async_copy(table_hbm.at[idx_v.at[b]], rows_v.at[b], gsem.at[b])

    def wait_gather(b):
        pltpu.make_async_copy(
            table_hbm.at[idx_v.at[b]], rows_v.at[b], gsem.at[b]).wait()

    def start_store(c, b):
        off = base + c * _CHUNK
        pltpu.async_copy(rows_v.at[b], out_hbm.at[pl.ds(off, _CHUNK)],
                         osem.at[b])

    def wait_store(c, b):
        off = base + c * _CHUNK
        pltpu.make_async_copy(
            rows_v.at[b], out_hbm.at[pl.ds(off, _CHUNK)], osem.at[b]).wait()

    for b in range(_NBUF):
        start_gather(b, b)

    def outer(o, carry):
        c0 = o * _NBUF
        for b in range(_NBUF):
            wait_gather(b)
            start_store(c0 + b, b)
        for b in range(_NBUF):
            wait_store(c0 + b, b)
            start_gather(c0 + b + _NBUF, b)
        return carry

    lax.fori_loop(0, _N_OUTER - 1, outer, 0)

    c0 = (_N_OUTER - 1) * _NBUF
    for b in range(_NBUF):
        wait_gather(b)
        start_store(c0 + b, b)
    for b in range(_NBUF):
        wait_store(c0 + b, b)


_emb = functools.partial(
    pl.kernel,
    out_type=jax.ShapeDtypeStruct((_B, D_MODEL), jnp.float32),
    mesh=plsc.VectorSubcoreMesh(core_axis_name="c", subcore_axis_name="s"),
    scratch_types=[
        pltpu.VMEM((_NBUF, _CHUNK), jnp.int32),
        pltpu.VMEM((_NBUF, _CHUNK, D_MODEL), jnp.float32),
        pltpu.SemaphoreType.DMA((_NBUF,)),
        pltpu.SemaphoreType.DMA((_NBUF,)),
    ],
    compiler_params=pltpu.CompilerParams(use_tc_tiling_on_sc=False),
)(_emb_body)


def kernel(sequence, lookup):
    idx = sequence.reshape(-1).astype(jnp.int32)
    out = _emb(idx, lookup)
    return out.reshape(sequence.shape + (D_MODEL,))
